# baseline (speedup 1.0000x reference)
import jax
import jax.numpy as jnp
from jax.experimental import pallas as pl
from jax.experimental.pallas import tpu as pltpu

B, L = 16384, 20
VOCAB, FEAT, D = 119, 200, 128
H = D // 2
BR = 256
GRID = B // BR
SLOTS = 4


def _table_kernel(cbfv_ref, projw_ref, bias_ref, out_ref):
    pt = jax.lax.dot_general(
        cbfv_ref[...], projw_ref[...],
        (((1,), (1,)), ((), ())),
        preferred_element_type=jnp.float32,
    )
    out_ref[...] = pt + bias_ref[...]


def _main_kernel(idx_ref, frac_ref, table_ref, w1_ref, b1_ref, w2t_ref,
                 out_hbm, scratch, sems):
    i = pl.program_id(0)
    slot = jax.lax.rem(i, SLOTS)

    @pl.when(i >= SLOTS)
    def _wait_prev():
        pltpu.make_async_copy(
            scratch.at[slot],
            out_hbm.at[pl.ds((i - SLOTS) * BR, BR)],
            sems.at[slot],
        ).wait()

    idx = idx_ref[...]
    frac = frac_ref[...]
    table = table_ref[...].astype(jnp.bfloat16)
    w2t = w2t_ref[...].astype(jnp.bfloat16)
    lane = jax.lax.broadcasted_iota(jnp.int32, (BR, 128), 1)
    for l in range(L):
        oh = (idx[:, l:l + 1] == lane).astype(jnp.bfloat16)
        h = frac[:, l:l + 1] * w1_ref[...] + b1_ref[...]
        h = (h * jax.nn.sigmoid(h)).astype(jnp.bfloat16)
        e = jnp.dot(oh, table, preferred_element_type=jnp.float32)
        f = jnp.dot(h, w2t, preferred_element_type=jnp.float32)
        scratch[slot, :, l * D:(l + 1) * D] = e + f

    pltpu.make_async_copy(
        scratch.at[slot],
        out_hbm.at[pl.ds(i * BR, BR)],
        sems.at[slot],
    ).start()

    nsteps = pl.num_programs(0)
    @pl.when(i == nsteps - 1)
    def _drain():
        for j in range(SLOTS):
            step = nsteps - SLOTS + j
            s = step % SLOTS
            pltpu.make_async_copy(
                scratch.at[s],
                out_hbm.at[pl.ds(step * BR, BR)],
                sems.at[s],
            ).wait()


def kernel(elem_idx, frac, cbfv_weight, proj_W, proj_b, mlp_W1, mlp_b1, mlp_W2, mlp_b2):
    cbfv_p = jnp.zeros((128, FEAT), cbfv_weight.dtype).at[:VOCAB].set(cbfv_weight)
    bias = (proj_b + mlp_b2).reshape(1, D)

    table = pl.pallas_call(
        _table_kernel,
        out_shape=jax.ShapeDtypeStruct((128, D), jnp.float32),
    )(cbfv_p, proj_W, bias)

    w1_row = mlp_W1.reshape(1, H)
    b1_row = mlp_b1.reshape(1, H)
    w2t = mlp_W2.T

    C = 4
    BC = B // C
    idx32 = elem_idx.astype(jnp.int32)
    chunks = []
    for c in range(C):
        o = pl.pallas_call(
            _main_kernel,
            grid=(BC // BR,),
            in_specs=[
                pl.BlockSpec((BR, L), lambda i: (i, 0)),
                pl.BlockSpec((BR, L), lambda i: (i, 0)),
                pl.BlockSpec((128, D), lambda i: (0, 0)),
                pl.BlockSpec((1, H), lambda i: (0, 0)),
                pl.BlockSpec((1, H), lambda i: (0, 0)),
                pl.BlockSpec((H, D), lambda i: (0, 0)),
            ],
            out_specs=pl.BlockSpec(memory_space=pl.ANY),
            out_shape=jax.ShapeDtypeStruct((BC, L * D), jnp.float32),
            scratch_shapes=[
                pltpu.VMEM((SLOTS, BR, L * D), jnp.float32),
                pltpu.SemaphoreType.DMA((SLOTS,)),
            ],
        )(idx32[c * BC:(c + 1) * BC], frac[c * BC:(c + 1) * BC],
          table, w1_row, b1_row, w2t)
        chunks.append(o.reshape(BC, L, D))

    return jnp.concatenate(chunks, axis=0)


# speedup vs baseline: 1.8504x; 1.8504x over previous
import jax
import jax.numpy as jnp
from jax.experimental import pallas as pl
from jax.experimental.pallas import tpu as pltpu

B, L = 16384, 20
VOCAB, FEAT, D = 119, 200, 128
H = D // 2
BR = 256
GRID = B // BR
SLOTS = 4


def _table_kernel(cbfv_ref, projw_ref, bias_ref, out_ref):
    pt = jax.lax.dot_general(
        cbfv_ref[...], projw_ref[...],
        (((1,), (1,)), ((), ())),
        preferred_element_type=jnp.float32,
    )
    out_ref[...] = pt + bias_ref[...]


def _main_kernel(idx_ref, frac_ref, table_ref, w1_ref, b1_ref, w2t_ref,
                 out_hbm, scratch, sems):
    i = pl.program_id(0)
    slot = jax.lax.rem(i, SLOTS)

    @pl.when(i >= SLOTS)
    def _wait_prev():
        pltpu.make_async_copy(
            scratch.at[slot],
            out_hbm.at[pl.ds((i - SLOTS) * BR, BR)],
            sems.at[slot],
        ).wait()

    idx = idx_ref[...]
    frac = frac_ref[...]
    table = table_ref[...].astype(jnp.bfloat16)
    w2t = w2t_ref[...].astype(jnp.bfloat16)
    lane = jax.lax.broadcasted_iota(jnp.int32, (BR, 128), 1)
    for l in range(L):
        oh = (idx[:, l:l + 1] == lane).astype(jnp.bfloat16)
        h = frac[:, l:l + 1] * w1_ref[...] + b1_ref[...]
        h = (h * jax.nn.sigmoid(h)).astype(jnp.bfloat16)
        e = jnp.dot(oh, table, preferred_element_type=jnp.float32)
        f = jnp.dot(h, w2t, preferred_element_type=jnp.float32)
        scratch[slot, :, l, :] = e + f

    pltpu.make_async_copy(
        scratch.at[slot],
        out_hbm.at[pl.ds(i * BR, BR)],
        sems.at[slot],
    ).start()

    @pl.when(i == GRID - 1)
    def _drain():
        for j in range(SLOTS):
            step = GRID - SLOTS + j
            s = step % SLOTS
            pltpu.make_async_copy(
                scratch.at[s],
                out_hbm.at[pl.ds(step * BR, BR)],
                sems.at[s],
            ).wait()


def kernel(elem_idx, frac, cbfv_weight, proj_W, proj_b, mlp_W1, mlp_b1, mlp_W2, mlp_b2):
    cbfv_p = jnp.zeros((128, FEAT), cbfv_weight.dtype).at[:VOCAB].set(cbfv_weight)
    bias = (proj_b + mlp_b2).reshape(1, D)

    table = pl.pallas_call(
        _table_kernel,
        out_shape=jax.ShapeDtypeStruct((128, D), jnp.float32),
    )(cbfv_p, proj_W, bias)

    w1_row = mlp_W1.reshape(1, H)
    b1_row = mlp_b1.reshape(1, H)
    w2t = mlp_W2.T

    return pl.pallas_call(
        _main_kernel,
        grid=(GRID,),
        in_specs=[
            pl.BlockSpec((BR, L), lambda i: (i, 0)),
            pl.BlockSpec((BR, L), lambda i: (i, 0)),
            pl.BlockSpec((128, D), lambda i: (0, 0)),
            pl.BlockSpec((1, H), lambda i: (0, 0)),
            pl.BlockSpec((1, H), lambda i: (0, 0)),
            pl.BlockSpec((H, D), lambda i: (0, 0)),
        ],
        out_specs=pl.BlockSpec(memory_space=pl.ANY),
        out_shape=jax.ShapeDtypeStruct((B, L, D), jnp.float32),
        scratch_shapes=[
            pltpu.VMEM((SLOTS, BR, L, D), jnp.float32),
            pltpu.SemaphoreType.DMA((SLOTS,)),
        ],
    )(elem_idx.astype(jnp.int32), frac, table, w1_row, b1_row, w2t)
